# baseline (speedup 1.0000x reference)
import functools

import jax
import jax.numpy as jnp
from jax import lax
from jax.experimental import pallas as pl
from jax.experimental.pallas import tpu as pltpu
from jax.experimental.pallas import tpu_sc as plsc

_B = 64
_NCLS = 13
_L = 16
_RPT = 16
_NT = _B // _RPT

_mesh = plsc.VectorSubcoreMesh(
    core_axis_name="c", subcore_axis_name="s", num_cores=1
)


@functools.partial(
    pl.kernel,
    mesh=_mesh,
    out_type=jax.ShapeDtypeStruct((_B, _NCLS), jnp.float32),
    scratch_types=[
        pltpu.VMEM((_L,), jnp.int32),
        pltpu.VMEM((_RPT, _NCLS), jnp.float32),
        pltpu.SemaphoreType.DMA,
    ],
    compiler_params=pltpu.CompilerParams(
        needs_layout_passes=False,
        skip_device_barrier=True,
        disable_semaphore_checks=True,
        disable_bounds_checks=True,
        use_tc_tiling_on_sc=False,
    ),
)
def _scatter_logits(labels_hbm, out_hbm, labels_v, buf_v, lsem):
    w = lax.axis_index("s")

    @pl.when(w < _NT)
    def _():
        lcp = pltpu.make_async_copy(
            labels_hbm.at[pl.ds(w * _L, _L)], labels_v, lsem
        )
        lcp.start()

        iota = lax.iota(jnp.int32, _L)
        neg = jnp.full((_L,), -10.0, jnp.float32)
        for c in range(_NCLS):
            plsc.store_scatter(
                buf_v, [iota, jnp.full((_L,), c, jnp.int32)], neg
            )
        lcp.wait()

        chunk = labels_v[:]
        ten = jnp.full((_L,), 10.0, jnp.float32)
        plsc.store_scatter(buf_v, [iota, chunk], ten)

        pltpu.sync_copy(buf_v, out_hbm.at[pl.ds(w * _RPT, _RPT)])


def kernel(x, labels):
    del x
    return _scatter_logits(labels)
